# baseline (speedup 1.0000x reference)
import functools

import jax
import jax.numpy as jnp
from jax.experimental import pallas as pl
from jax.experimental.pallas import tpu as pltpu

_ALPHA = 0.0001
_NEG = -1e30
_LANES = 128


def _router_body(x_ref, w_ref, b_ref, gates_ref, topi_ref, loss_ref, cnt_ref,
                 *, n_experts, top_k, alpha):
    T = x_ref.shape[0]
    L = _LANES
    x = x_ref[...]
    logits = jnp.dot(x, w_ref[...], preferred_element_type=jnp.float32)
    lane = jax.lax.broadcasted_iota(jnp.int32, (T, L), 1)
    valid = lane < n_experts
    aff = jnp.where(valid, jax.nn.sigmoid(logits), 0.0)
    scores = aff + b_ref[...]
    m1 = jnp.max(scores, axis=1, keepdims=True)
    i1 = jnp.min(jnp.where(scores == m1, lane, L), axis=1, keepdims=True)
    g1 = jnp.sum(jnp.where(lane == i1, aff, 0.0), axis=1, keepdims=True)
    scores2 = jnp.where(lane == i1, _NEG, scores)
    m2 = jnp.max(scores2, axis=1, keepdims=True)
    i2 = jnp.min(jnp.where(scores2 == m2, lane, L), axis=1, keepdims=True)
    g2 = jnp.sum(jnp.where(lane == i2, aff, 0.0), axis=1, keepdims=True)
    denom = g1 + g2 + 1e-9
    w1 = g1 / denom
    w2 = g2 / denom
    rowsum = jnp.sum(aff, axis=1, keepdims=True)
    pvec = jnp.sum(aff / (rowsum + 1e-9), axis=0) / T
    cnt = jnp.sum((lane == i1).astype(jnp.int32) + (lane == i2).astype(jnp.int32),
                  axis=0)
    f = cnt.astype(jnp.float32) * (n_experts / (top_k * T))
    loss = alpha * jnp.sum(f * pvec)
    gates_ref[...] = jnp.where(lane == 0, w1, jnp.where(lane == 1, w2, 0.0))
    topi_ref[...] = jnp.where(lane == 0, i1, jnp.where(lane == 1, i2, 0))
    loss_ref[...] = jnp.full(loss_ref.shape, loss, jnp.float32)
    cnt_ref[...] = jnp.broadcast_to(cnt[None, :], cnt_ref.shape)


def _swiglu_body(x_ref, g_ref, u_ref, d_ref, o_ref):
    x = x_ref[...]
    g = jnp.dot(x, g_ref[...], preferred_element_type=jnp.float32)
    u = jnp.dot(x, u_ref[...], preferred_element_type=jnp.float32)
    h = ((g * jax.nn.sigmoid(g)) * u).astype(jnp.bfloat16)
    o_ref[...] = jnp.dot(h, d_ref[...], preferred_element_type=jnp.float32)


def _group_body(be_ref, x_ref, g_ref, u_ref, d_ref, o_ref):
    del be_ref
    x = x_ref[...]
    g = jnp.dot(x, g_ref[0], preferred_element_type=jnp.float32)
    u = jnp.dot(x, u_ref[0], preferred_element_type=jnp.float32)
    h = ((g * jax.nn.sigmoid(g)) * u).astype(jnp.bfloat16)
    o_ref[...] = jnp.dot(h, d_ref[0], preferred_element_type=jnp.float32)


def kernel(x, s_gate, s_up, s_down, e_gate, e_up, e_down, router_w, expert_bias):
    B, S, D = x.shape
    E, _, I_R = e_gate.shape
    I_S = s_gate.shape[1]
    K = 2
    T = B * S
    L = _LANES

    xf = x.reshape(T, D)
    xb = xf.astype(jnp.bfloat16)
    sg_b = s_gate.astype(jnp.bfloat16)
    su_b = s_up.astype(jnp.bfloat16)
    sd_b = s_down.astype(jnp.bfloat16)
    eg_b = e_gate.astype(jnp.bfloat16)
    eu_b = e_up.astype(jnp.bfloat16)
    ed_b = e_down.astype(jnp.bfloat16)

    w_pad = jnp.zeros((D, L), jnp.float32).at[:, :E].set(router_w)
    b_pad = jnp.full((1, L), _NEG, jnp.float32).at[0, :E].set(expert_bias)
    gates128, topi128, loss128, cnt128 = pl.pallas_call(
        functools.partial(_router_body, n_experts=E, top_k=K, alpha=_ALPHA),
        out_shape=(
            jax.ShapeDtypeStruct((T, L), jnp.float32),
            jax.ShapeDtypeStruct((T, L), jnp.int32),
            jax.ShapeDtypeStruct((8, L), jnp.float32),
            jax.ShapeDtypeStruct((8, L), jnp.int32),
        ),
    )(xf, w_pad, b_pad)
    balance_loss = loss128[0, 0]
    counts = cnt128[0, :E]

    BLK = 256 if T * K >= 4096 else max(8, (T * K) // 8)
    PAD = T * K + E * BLK
    NB = PAD // BLK

    topi_tk = topi128[:, :K]
    gates_tk = gates128[:, :K]
    ea = topi_tk.reshape(-1)
    oh = (ea[:, None] == jnp.arange(E, dtype=ea.dtype)[None, :]).astype(jnp.int32)
    ranks = jnp.cumsum(oh, axis=0) - oh
    r_sel = jnp.take_along_axis(ranks, ea[:, None], axis=1)[:, 0]
    padded = ((counts + BLK - 1) // BLK) * BLK
    pstart = jnp.concatenate([jnp.zeros((1,), jnp.int32),
                              jnp.cumsum(padded)[:-1].astype(jnp.int32)])
    dest = pstart[ea] + r_sel

    tok = (jnp.arange(T * K, dtype=jnp.int32) // K).astype(jnp.int32)
    row_id = jnp.zeros((PAD,), jnp.int32).at[dest].set(
        tok, mode="drop", unique_indices=True)
    p = jnp.arange(NB, dtype=jnp.int32) * BLK
    pend = pstart + padded
    block_expert = jnp.minimum(
        jnp.sum((p[:, None] >= pend[None, :]).astype(jnp.int32), axis=1), E - 1)

    xg = xb[row_id]
    yg = pl.pallas_call(
        _group_body,
        grid_spec=pltpu.PrefetchScalarGridSpec(
            num_scalar_prefetch=1,
            grid=(NB,),
            in_specs=[
                pl.BlockSpec((BLK, D), lambda b, be: (b, 0)),
                pl.BlockSpec((1, D, I_R), lambda b, be: (be[b], 0, 0)),
                pl.BlockSpec((1, D, I_R), lambda b, be: (be[b], 0, 0)),
                pl.BlockSpec((1, I_R, D), lambda b, be: (be[b], 0, 0)),
            ],
            out_specs=pl.BlockSpec((BLK, D), lambda b, be: (b, 0)),
        ),
        out_shape=jax.ShapeDtypeStruct((PAD, D), jnp.float32),
    )(block_expert, xg, eg_b, eu_b, ed_b)

    BT = min(256, T)
    shared = pl.pallas_call(
        _swiglu_body,
        grid=(T // BT,),
        in_specs=[
            pl.BlockSpec((BT, D), lambda b: (b, 0)),
            pl.BlockSpec((D, I_S), lambda b: (0, 0)),
            pl.BlockSpec((D, I_S), lambda b: (0, 0)),
            pl.BlockSpec((I_S, D), lambda b: (0, 0)),
        ],
        out_specs=pl.BlockSpec((BT, D), lambda b: (b, 0)),
        out_shape=jax.ShapeDtypeStruct((T, D), jnp.float32),
    )(xb, sg_b, su_b, sd_b)

    dmat = dest.reshape(T, K)
    routed = (gates_tk[:, :1] * yg[dmat[:, 0]] +
              gates_tk[:, 1:2] * yg[dmat[:, 1]])
    output = (shared + routed).reshape(B, S, D)
    expert_counts = counts.astype(jnp.int32)
    return (output, balance_loss, expert_counts)


# speedup vs baseline: 1.1162x; 1.1162x over previous
import functools

import jax
import jax.numpy as jnp
from jax.experimental import pallas as pl
from jax.experimental.pallas import tpu as pltpu

_ALPHA = 0.0001
_NEG = -1e30
_LANES = 128


def _router_body(x_ref, w_ref, b_ref, gates_ref, topi_ref, loss_ref, cnt_ref,
                 *, n_experts, top_k, alpha):
    T = x_ref.shape[0]
    L = _LANES
    x = x_ref[...]
    logits = jnp.dot(x, w_ref[...], preferred_element_type=jnp.float32)
    lane = jax.lax.broadcasted_iota(jnp.int32, (T, L), 1)
    valid = lane < n_experts
    aff = jnp.where(valid, jax.nn.sigmoid(logits), 0.0)
    scores = aff + b_ref[...]
    m1 = jnp.max(scores, axis=1, keepdims=True)
    i1 = jnp.min(jnp.where(scores == m1, lane, L), axis=1, keepdims=True)
    g1 = jnp.sum(jnp.where(lane == i1, aff, 0.0), axis=1, keepdims=True)
    scores2 = jnp.where(lane == i1, _NEG, scores)
    m2 = jnp.max(scores2, axis=1, keepdims=True)
    i2 = jnp.min(jnp.where(scores2 == m2, lane, L), axis=1, keepdims=True)
    g2 = jnp.sum(jnp.where(lane == i2, aff, 0.0), axis=1, keepdims=True)
    denom = g1 + g2 + 1e-9
    w1 = g1 / denom
    w2 = g2 / denom
    rowsum = jnp.sum(aff, axis=1, keepdims=True)
    pvec = jnp.sum(aff / (rowsum + 1e-9), axis=0) / T
    cnt = jnp.sum((lane == i1).astype(jnp.int32) + (lane == i2).astype(jnp.int32),
                  axis=0)
    f = cnt.astype(jnp.float32) * (n_experts / (top_k * T))
    loss = alpha * jnp.sum(f * pvec)
    gates_ref[...] = jnp.where(lane == 0, w1, jnp.where(lane == 1, w2, 0.0))
    topi_ref[...] = jnp.where(lane == 0, i1, jnp.where(lane == 1, i2, 0))
    loss_ref[...] = jnp.full(loss_ref.shape, loss, jnp.float32)
    cnt_ref[...] = jnp.broadcast_to(cnt[None, :], cnt_ref.shape)


def _swiglu_body(x_ref, g_ref, u_ref, d_ref, o_ref):
    x = x_ref[...]
    g = jnp.dot(x, g_ref[...], preferred_element_type=jnp.float32)
    u = jnp.dot(x, u_ref[...], preferred_element_type=jnp.float32)
    h = ((g * jax.nn.sigmoid(g)) * u).astype(jnp.bfloat16)
    o_ref[...] = jnp.dot(h, d_ref[...], preferred_element_type=jnp.float32)


def _group_body(be_ref, x_ref, g_ref, u_ref, d_ref, o_ref):
    del be_ref
    x = x_ref[...]
    g = jnp.dot(x, g_ref[0], preferred_element_type=jnp.float32)
    u = jnp.dot(x, u_ref[0], preferred_element_type=jnp.float32)
    h = ((g * jax.nn.sigmoid(g)) * u).astype(jnp.bfloat16)
    o_ref[...] = jnp.dot(h, d_ref[0], preferred_element_type=jnp.float32)


def kernel(x, s_gate, s_up, s_down, e_gate, e_up, e_down, router_w, expert_bias):
    B, S, D = x.shape
    E, _, I_R = e_gate.shape
    I_S = s_gate.shape[1]
    K = 2
    T = B * S
    L = _LANES

    xf = x.reshape(T, D)

    w_pad = jnp.zeros((D, L), jnp.float32).at[:, :E].set(router_w)
    b_pad = jnp.full((1, L), _NEG, jnp.float32).at[0, :E].set(expert_bias)
    gates128, topi128, loss128, cnt128 = pl.pallas_call(
        functools.partial(_router_body, n_experts=E, top_k=K, alpha=_ALPHA),
        out_shape=(
            jax.ShapeDtypeStruct((T, L), jnp.float32),
            jax.ShapeDtypeStruct((T, L), jnp.int32),
            jax.ShapeDtypeStruct((8, L), jnp.float32),
            jax.ShapeDtypeStruct((8, L), jnp.int32),
        ),
    )(xf, w_pad, b_pad)
    balance_loss = loss128[0, 0]
    counts = cnt128[0, :E]

    BLK = 256 if T * K >= 4096 else max(8, (T * K) // 8)
    PAD = T * K + E * BLK
    NB = PAD // BLK

    topi_tk = topi128[:, :K]
    gates_tk = gates128[:, :K]
    ea = topi_tk.reshape(-1)
    oh = (ea[:, None] == jnp.arange(E, dtype=ea.dtype)[None, :]).astype(jnp.int32)
    ranks = jnp.cumsum(oh, axis=0) - oh
    r_sel = jnp.take_along_axis(ranks, ea[:, None], axis=1)[:, 0]
    padded = ((counts + BLK - 1) // BLK) * BLK
    pstart = jnp.concatenate([jnp.zeros((1,), jnp.int32),
                              jnp.cumsum(padded)[:-1].astype(jnp.int32)])
    dest = pstart[ea] + r_sel

    tok = (jnp.arange(T * K, dtype=jnp.int32) // K).astype(jnp.int32)
    row_id = jnp.zeros((PAD,), jnp.int32).at[dest].set(
        tok, mode="promise_in_bounds", unique_indices=True)
    p = jnp.arange(NB, dtype=jnp.int32) * BLK
    pend = pstart + padded
    block_expert = jnp.minimum(
        jnp.sum((p[:, None] >= pend[None, :]).astype(jnp.int32), axis=1), E - 1)

    xg = xf.at[row_id].get(mode="promise_in_bounds")
    yg = pl.pallas_call(
        _group_body,
        grid_spec=pltpu.PrefetchScalarGridSpec(
            num_scalar_prefetch=1,
            grid=(NB,),
            in_specs=[
                pl.BlockSpec((BLK, D), lambda b, be: (b, 0)),
                pl.BlockSpec((1, D, I_R), lambda b, be: (be[b], 0, 0)),
                pl.BlockSpec((1, D, I_R), lambda b, be: (be[b], 0, 0)),
                pl.BlockSpec((1, I_R, D), lambda b, be: (be[b], 0, 0)),
            ],
            out_specs=pl.BlockSpec((BLK, D), lambda b, be: (b, 0)),
        ),
        out_shape=jax.ShapeDtypeStruct((PAD, D), jnp.float32),
    )(block_expert, xg, e_gate, e_up, e_down)

    BT = min(256, T)
    shared = pl.pallas_call(
        _swiglu_body,
        grid=(T // BT,),
        in_specs=[
            pl.BlockSpec((BT, D), lambda b: (b, 0)),
            pl.BlockSpec((D, I_S), lambda b: (0, 0)),
            pl.BlockSpec((D, I_S), lambda b: (0, 0)),
            pl.BlockSpec((I_S, D), lambda b: (0, 0)),
        ],
        out_specs=pl.BlockSpec((BT, D), lambda b: (b, 0)),
        out_shape=jax.ShapeDtypeStruct((T, D), jnp.float32),
    )(xf, s_gate, s_up, s_down)

    dmat = dest.reshape(T, K)
    routed = (gates_tk[:, :1] * yg.at[dmat[:, 0]].get(mode="promise_in_bounds") +
              gates_tk[:, 1:2] * yg.at[dmat[:, 1]].get(mode="promise_in_bounds"))
    output = (shared + routed).reshape(B, S, D)
    expert_counts = counts.astype(jnp.int32)
    return (output, balance_loss, expert_counts)


# speedup vs baseline: 1.5441x; 1.3833x over previous
import functools

import jax
import jax.numpy as jnp
from jax import lax
from jax.experimental import pallas as pl
from jax.experimental.pallas import tpu as pltpu
from jax.experimental.pallas import tpu_sc as plsc

_ALPHA = 0.0001
_NEG = -1e30
_LANES = 128


def _router_body(x_ref, w_ref, b_ref, gates_ref, topi_ref, loss_ref, cnt_ref,
                 *, n_experts, top_k, alpha):
    T = x_ref.shape[0]
    E = n_experts
    x = x_ref[...]
    logits = jnp.dot(x, w_ref[...], preferred_element_type=jnp.float32)
    lane = jax.lax.broadcasted_iota(jnp.int32, (T, E), 1)
    aff = jax.nn.sigmoid(logits)
    scores = aff + b_ref[...]
    m1 = jnp.max(scores, axis=1, keepdims=True)
    i1 = jnp.min(jnp.where(scores == m1, lane, E), axis=1, keepdims=True)
    g1 = jnp.sum(jnp.where(lane == i1, aff, 0.0), axis=1, keepdims=True)
    scores2 = jnp.where(lane == i1, _NEG, scores)
    m2 = jnp.max(scores2, axis=1, keepdims=True)
    i2 = jnp.min(jnp.where(scores2 == m2, lane, E), axis=1, keepdims=True)
    g2 = jnp.sum(jnp.where(lane == i2, aff, 0.0), axis=1, keepdims=True)
    denom = g1 + g2 + 1e-9
    w1 = g1 / denom
    w2 = g2 / denom
    rowsum = jnp.sum(aff, axis=1, keepdims=True)
    pvec = jnp.sum(aff / (rowsum + 1e-9), axis=0) / T
    cnt = jnp.sum((lane == i1).astype(jnp.int32) + (lane == i2).astype(jnp.int32),
                  axis=0)
    f = cnt.astype(jnp.float32) * (n_experts / (top_k * T))
    loss = alpha * jnp.sum(f * pvec)
    gates_ref[...] = jnp.where(lane == 0, w1, jnp.where(lane == 1, w2, 0.0))
    topi_ref[...] = jnp.where(lane == 0, i1, jnp.where(lane == 1, i2, 0))
    loss_ref[...] = jnp.full(loss_ref.shape, loss, jnp.float32)
    cnt_ref[...] = jnp.broadcast_to(cnt[None, :], cnt_ref.shape)


def _vsum(x, lanes):
    for sh in (8, 4, 2, 1):
        idx = lanes + sh
        idx = jnp.where(idx >= 16, idx - 16, idx)
        x = x + x.at[idx].get(mode="promise_in_bounds")
    return x


def _vprefix(x, lanes):
    for sh in (1, 2, 4, 8):
        idx = lanes - sh
        idx = jnp.where(idx < 0, 0, idx)
        sh_x = x.at[idx].get(mode="promise_in_bounds")
        x = x + jnp.where(lanes >= sh, sh_x, 0)
    return x


def _dispatch_body(ea_hbm, x_hbm, dest_hbm, beb_hbm, xg_hbm,
                   ea_v, row_v, row_v2, dst_v, bev_ref, sem, sem_s1, sem_s2,
                   *, n_experts, blk, n_tokens):
    TK = ea_v.shape[0]
    NW = 32
    CH = TK // NW
    CHV = CH // 16
    NV = TK // 16
    E8 = n_experts
    w = lax.axis_index("s") * 2 + lax.axis_index("c")

    pltpu.sync_copy(ea_hbm, ea_v)
    lanes = lax.iota(jnp.int32, 16)
    zero16 = jnp.zeros((16,), jnp.int32)

    def count_step(i, acc):
        vals = ea_v[pl.ds(i * 16, 16)]
        return tuple(acc[e] + jnp.where(vals == e, 1, 0)
                     for e in range(E8))

    wv = w * CHV
    init = (zero16,) * E8
    acc_pre = lax.fori_loop(0, wv, count_step, init)
    acc_post = lax.fori_loop(wv, NV, count_step, init)

    pre = zero16
    tot = zero16
    for e in range(E8):
        pe = _vsum(acc_pre[e], lanes)
        te = pe + _vsum(acc_post[e], lanes)
        sel = jnp.where(lanes == e, 1, 0)
        pre = pre + sel * pe
        tot = tot + sel * te

    padded = (tot + (blk - 1)) & ~(blk - 1)
    pstart = _vprefix(padded, lanes) - padded
    off = pstart + pre

    @pl.when(w == 0)
    def _():
        pend = pstart + padded
        for k in range(3):
            bstart = (lanes + k * 16) * blk
            be = jnp.zeros((16,), jnp.int32)
            for e in range(n_experts):
                pe = _vsum(jnp.where(lanes == e, pend, 0), lanes)
                be = be + jnp.where(bstart >= pe, 1, 0)
            bev_ref[pl.ds(k * 16, 16)] = jnp.minimum(be, n_experts - 1)
        pltpu.sync_copy(bev_ref, beb_hbm)

    for v in range(CHV):
        vals = ea_v[pl.ds((wv + v) * 16, 16)]
        dvals = zero16
        for e in range(E8):
            m = vals == e
            mi = jnp.where(m, 1, 0)
            cs = _vprefix(mi, lanes)
            off_e = _vsum(jnp.where(lanes == e, off, 0), lanes)
            dvals = jnp.where(m, off_e + cs - 1, dvals)
            off = off + jnp.where(lanes == e, _vsum(mi, lanes), 0)
        dst_v[v // 2, pl.ds((v % 2) * 16, 16)] = dvals

    pltpu.sync_copy(dst_v, dest_hbm.at[pl.ds(w * (CH // 32), CH // 32)])

    t0 = (w % (n_tokens // CH)) * CH
    NQ = CH // 32
    row_bufs = (row_v, row_v2)
    ssems = (sem_s1, sem_s2)

    def _load(q, buf):
        return pltpu.async_copy(x_hbm.at[pl.ds(t0 + q * 32, 32)], buf, sem)

    cur = _load(0, row_bufs[0])
    prev_st = [None, None]
    for q in range(NQ):
        buf = row_bufs[q % 2]
        cur.wait()
        if q + 1 < NQ:
            nxt_buf = row_bufs[(q + 1) % 2]
            if prev_st[(q + 1) % 2] is not None:
                prev_st[(q + 1) % 2].wait()
                prev_st[(q + 1) % 2] = None
            cur = _load(q + 1, nxt_buf)
        prev_st[q % 2] = pltpu.async_copy(buf, xg_hbm.at[dst_v.at[q]],
                                          ssems[q % 2])
    for st in prev_st:
        if st is not None:
            st.wait()


def _swiglu_body(x_ref, g_ref, u_ref, d_ref, o_ref):
    x = x_ref[...]
    g = jnp.dot(x, g_ref[...], preferred_element_type=jnp.float32)
    u = jnp.dot(x, u_ref[...], preferred_element_type=jnp.float32)
    h = ((g * jax.nn.sigmoid(g)) * u).astype(jnp.bfloat16)
    o_ref[...] = jnp.dot(h, d_ref[...], preferred_element_type=jnp.float32)


def _group_body(be_ref, x_ref, g_ref, u_ref, d_ref, o_ref):
    del be_ref
    x = x_ref[...]
    g = jnp.dot(x, g_ref[0], preferred_element_type=jnp.float32)
    u = jnp.dot(x, u_ref[0], preferred_element_type=jnp.float32)
    h = ((g * jax.nn.sigmoid(g)) * u).astype(jnp.bfloat16)
    o_ref[...] = jnp.dot(h, d_ref[0], preferred_element_type=jnp.float32)


def kernel(x, s_gate, s_up, s_down, e_gate, e_up, e_down, router_w, expert_bias):
    B, S, D = x.shape
    E, _, I_R = e_gate.shape
    I_S = s_gate.shape[1]
    K = 2
    T = B * S
    L = _LANES

    xf = x.reshape(T, D)

    gates128, topi128, loss128, cnt128 = pl.pallas_call(
        functools.partial(_router_body, n_experts=E, top_k=K, alpha=_ALPHA),
        out_shape=(
            jax.ShapeDtypeStruct((T, E), jnp.float32),
            jax.ShapeDtypeStruct((T, E), jnp.int32),
            jax.ShapeDtypeStruct((8, E), jnp.float32),
            jax.ShapeDtypeStruct((8, E), jnp.int32),
        ),
    )(xf, router_w, expert_bias.reshape(1, E))
    balance_loss = loss128[0, 0]
    counts = cnt128[0, :E]

    BLK = 256
    TK = T * K
    PAD = TK + E * BLK
    NB = PAD // BLK

    ea2 = jnp.concatenate([topi128[:, 0], topi128[:, 1]])
    mesh = plsc.VectorSubcoreMesh(core_axis_name="c", subcore_axis_name="s")
    dest2, beb, xg = pl.kernel(
        functools.partial(_dispatch_body, n_experts=E, blk=BLK, n_tokens=T),
        out_type=(
            jax.ShapeDtypeStruct((TK // 32, 32), jnp.int32),
            jax.ShapeDtypeStruct((48,), jnp.int32),
            jax.ShapeDtypeStruct((PAD, D), jnp.float32),
        ),
        mesh=mesh,
        scratch_types=[
            pltpu.VMEM((TK,), jnp.int32),
            pltpu.VMEM((32, D), jnp.float32),
            pltpu.VMEM((32, D), jnp.float32),
            pltpu.VMEM((TK // 32 // 32, 32), jnp.int32),
            pltpu.VMEM((48,), jnp.int32),
            pltpu.SemaphoreType.DMA,
            pltpu.SemaphoreType.DMA,
            pltpu.SemaphoreType.DMA,
        ],
    )(ea2, xf)
    dest = dest2.reshape(-1)

    yg = pl.pallas_call(
        _group_body,
        grid_spec=pltpu.PrefetchScalarGridSpec(
            num_scalar_prefetch=1,
            grid=(NB,),
            in_specs=[
                pl.BlockSpec((BLK, D), lambda b, be: (b, 0)),
                pl.BlockSpec((1, D, I_R), lambda b, be: (be[b], 0, 0)),
                pl.BlockSpec((1, D, I_R), lambda b, be: (be[b], 0, 0)),
                pl.BlockSpec((1, I_R, D), lambda b, be: (be[b], 0, 0)),
            ],
            out_specs=pl.BlockSpec((BLK, D), lambda b, be: (b, 0)),
        ),
        out_shape=jax.ShapeDtypeStruct((PAD, D), jnp.float32),
    )(beb, xg, e_gate, e_up, e_down)

    BT = min(256, T)
    shared = pl.pallas_call(
        _swiglu_body,
        grid=(T // BT,),
        in_specs=[
            pl.BlockSpec((BT, D), lambda b: (b, 0)),
            pl.BlockSpec((D, I_S), lambda b: (0, 0)),
            pl.BlockSpec((D, I_S), lambda b: (0, 0)),
            pl.BlockSpec((I_S, D), lambda b: (0, 0)),
        ],
        out_specs=pl.BlockSpec((BT, D), lambda b: (b, 0)),
        out_shape=jax.ShapeDtypeStruct((T, D), jnp.float32),
    )(xf, s_gate, s_up, s_down)

    ygall = yg.at[dest].get(mode="promise_in_bounds")
    routed = (gates128[:, :1] * ygall[:T] +
              gates128[:, 1:2] * ygall[T:])
    output = (shared + routed).reshape(B, S, D)
    expert_counts = counts.astype(jnp.int32)
    return (output, balance_loss, expert_counts)
